# Initial kernel scaffold; baseline (speedup 1.0000x reference)
import jax
import jax.numpy as jnp
from jax.experimental import pallas as pl


def kernel(x, edge_index, Wg0, Wg12, bg, Wa, a_src, a_dst, ba, Wout, bout):
    raise NotImplementedError("write your pallas kernel here")


# speedup vs baseline: 8.7185x; 8.7185x over previous
import functools

import jax
import jax.numpy as jnp
from jax import lax
from jax.experimental import pallas as pl
from jax.experimental.pallas import tpu as pltpu
from jax.experimental.pallas import tpu_sc as plsc

NN = 10000
NC7 = 7
EE = 320000
DD = 128
HH = 8
CC = 16
NCORE = 1
NSUB = 16
NWORK = NCORE * NSUB
EPW = EE // NWORK
K = 32
NCHUNK = EPW // K
CH = 80
ZU = 16
NZN = NN // ZU
NPACK = 1280

_f32 = jnp.float32
_i32 = jnp.int32

_GDN = lax.GatherDimensionNumbers(
    offset_dims=(), collapsed_slice_dims=(0,), start_index_map=(0,))


def _lane_bcast(v, h):
    idx = jnp.full((16, 1), h, _i32)
    return lax.gather(v, idx, _GDN, (1,),
                      mode=lax.GatherScatterMode.PROMISE_IN_BOUNDS)


def _zero_rows(zb, width):
    def body(i, carry):
        for j in range(width // 16):
            zb[i, pl.ds(j * 16, 16)] = jnp.zeros((16,), _f32)
        return carry
    lax.fori_loop(0, K, body, 0)


def _strided_units(s, nunits, fn):
    nper = -(-nunits // NSUB)

    def body(t, carry):
        cid = s + NSUB * t
        @pl.when(cid < nunits)
        def _():
            fn(cid)
        return carry
    lax.fori_loop(0, nper, body, 0)


def _wid(c, s):
    return c * NSUB + s


def _sc_deg_body(dst_hbm, out_hbm, didx, dpack, ones, acc):
    c = lax.axis_index("c")
    s = lax.axis_index("s")
    _zero_rows(ones, DD)

    lane = lax.iota(_i32, 16)
    one_row = jnp.where(lane == 0, 1.0, 0.0).astype(_f32)
    zrow = jnp.zeros((16,), _f32)

    _strided_units(s, NPACK // ZU, lambda u: pltpu.sync_copy(
        ones.at[pl.ds(0, ZU)], acc.at[pl.ds(u * ZU, ZU)]))
    plsc.subcore_barrier()

    base0 = _wid(c, s) * EPW

    def chunk(g, carry):
        pltpu.sync_copy(dst_hbm.at[pl.ds(base0 + g * K, K)], didx)

        def grp(v, carry2):
            dv = didx[pl.ds(v * 16, 16)]
            dpack[pl.ds(v * 16, 16)] = lax.shift_right_logical(dv, 3)
            offv = (dv & 7) * 16
            for l in range(16):
                e = v * 16 + l
                off = offv[l]
                for j in range(DD // 16):
                    ones[e, pl.ds(j * 16, 16)] = zrow
                ones[e, pl.ds(off, 16)] = one_row
            return carry2
        lax.fori_loop(0, K // 16, grp, 0)

        pltpu.sync_copy(ones, acc.at[dpack], add=True)
        return carry
    lax.fori_loop(0, NCHUNK, chunk, 0)

    plsc.subcore_barrier()
    pltpu.sync_copy(acc.at[pl.ds(s * CH, CH)], out_hbm.at[pl.ds(s * CH, CH)])


def _sc_edge_body(hs_hbm, vm_hbm, src_hbm, dst_hbm, num_hbm, den_hbm,
                  sidx, didx, dpack, hsbuf, vmbuf, sbuf, exbuf,
                  accn, accd, sem):
    c = lax.axis_index("c")
    s = lax.axis_index("s")
    _zero_rows(sbuf, DD)

    _strided_units(s, NZN, lambda u: pltpu.sync_copy(
        sbuf.at[pl.ds(0, ZU)], accn.at[pl.ds(u * ZU, ZU)]))
    _strided_units(s, NPACK // ZU, lambda u: pltpu.sync_copy(
        sbuf.at[pl.ds(0, ZU)], accd.at[pl.ds(u * ZU, ZU)]))
    plsc.subcore_barrier()

    base0 = _wid(c, s) * EPW
    zrow = jnp.zeros((16,), _f32)

    def chunk(g, carry):
        base = base0 + g * K
        pltpu.sync_copy(src_hbm.at[pl.ds(base, K)], sidx)
        pltpu.sync_copy(dst_hbm.at[pl.ds(base, K)], didx)
        cp_hs = pltpu.async_copy(hs_hbm.at[sidx], hsbuf, sem)
        cp_vm = pltpu.async_copy(vm_hbm.at[didx], vmbuf, sem)
        cp_hs.wait()
        cp_vm.wait()

        def grp(v, carry2):
            dv = didx[pl.ds(v * 16, 16)]
            dpack[pl.ds(v * 16, 16)] = lax.shift_right_logical(dv, 3)
            offv = (dv & 7) * 16
            for l in range(16):
                e = v * 16 + l
                off = offv[l]
                u = hsbuf[e, pl.ds(DD, 16)]
                edv = vmbuf[e, pl.ds(0, 16)]
                mm = vmbuf[e, pl.ds(16, 16)]
                z = u + edv
                lr = jnp.where(z > 0, z, 0.2 * z)
                ex = jnp.exp(lr - mm)
                for j in range(DD // 16):
                    exbuf[e, pl.ds(j * 16, 16)] = zrow
                exbuf[e, pl.ds(off, 16)] = ex
                for h in range(HH):
                    b = _lane_bcast(ex, h)
                    sbuf[e, pl.ds(h * 16, 16)] = (
                        hsbuf[e, pl.ds(h * 16, 16)] * b)
            return carry2
        lax.fori_loop(0, K // 16, grp, 0)

        pltpu.sync_copy(sbuf, accn.at[didx], add=True)
        pltpu.sync_copy(exbuf, accd.at[dpack], add=True)
        return carry
    lax.fori_loop(0, NCHUNK, chunk, 0)

    plsc.subcore_barrier()
    _strided_units(s, NZN, lambda u: pltpu.sync_copy(
        accn.at[pl.ds(u * ZU, ZU)], num_hbm.at[pl.ds(u * ZU, ZU)]))
    pltpu.sync_copy(accd.at[pl.ds(s * CH, CH)],
                    den_hbm.at[pl.ds(s * CH, CH)])


@functools.cache
def _sc_kernels():
    mesh = plsc.VectorSubcoreMesh(
        core_axis_name="c", subcore_axis_name="s",
        num_cores=NCORE, num_subcores=NSUB)
    deg = pl.kernel(
        _sc_deg_body,
        out_type=jax.ShapeDtypeStruct((NPACK, DD), _f32),
        mesh=mesh,
        scratch_types=[
            pltpu.VMEM((K,), _i32),
            pltpu.VMEM((K,), _i32),
            pltpu.VMEM((K, DD), _f32),
            pltpu.VMEM_SHARED((NPACK, DD), _f32),
        ])
    edge = pl.kernel(
        _sc_edge_body,
        out_type=[jax.ShapeDtypeStruct((NN, DD), _f32),
                  jax.ShapeDtypeStruct((NPACK, DD), _f32)],
        mesh=mesh,
        scratch_types=[
            pltpu.VMEM((K,), _i32),
            pltpu.VMEM((K,), _i32),
            pltpu.VMEM((K,), _i32),
            pltpu.VMEM((K, 2 * DD), _f32),
            pltpu.VMEM((K, DD), _f32),
            pltpu.VMEM((K, DD), _f32),
            pltpu.VMEM((K, DD), _f32),
            pltpu.VMEM_SHARED((NN, DD), _f32),
            pltpu.VMEM_SHARED((NPACK, DD), _f32),
            pltpu.SemaphoreType.DMA,
        ])
    return deg, edge


BN = 1000
GRID = NN // BN


def _a0_body(x_ref, w_ref, dinv_ref, o_ref):
    h1 = jnp.dot(x_ref[...], w_ref[...],
                 preferred_element_type=_f32) * dinv_ref[...]
    o_ref[...] = jnp.concatenate([h1, jnp.zeros_like(h1)], axis=1)


def _c_body(p0, h1p, dinv, bgr, wa, ms, md, hso, eso, edo, mo):
    hg = jnp.maximum(
        (p0[...] + h1p[...][:, :DD]) * dinv[...] + bgr[...], 0.0)
    h2 = jnp.dot(hg, wa[...], preferred_element_type=_f32)
    es = jnp.dot(h2, ms[...], preferred_element_type=_f32)
    ed = jnp.dot(h2, md[...], preferred_element_type=_f32)
    hso[...] = jnp.concatenate(
        [h2, es, es, jnp.zeros((h2.shape[0], 2 * DD - DD - 2 * HH), _f32)],
        axis=1)
    eso[...] = es
    edo[...] = ed
    mo[...] = jnp.max(es, axis=0, keepdims=True)[None]


def _e_common(n0, d0, hs, es, ed, mg, rr, ss, bar):
    esv = es[...]
    edv = ed[...]
    z = esv + edv
    lr = jnp.where(z > 0, z, 0.2 * z)
    mp = jnp.maximum(0.0, mg[...] + edv)
    exs = jnp.exp(lr - mp)
    den = d0[...][:, :HH] + exs
    num = n0[...] + jnp.dot(exs, rr[...],
                            preferred_element_type=_f32) * hs[...][:, :DD]
    denw = jnp.dot(den, rr[...], preferred_element_type=_f32) + 1e-16
    return jnp.dot(num / denw, ss[...], preferred_element_type=_f32) + bar[...]


def _e_body(n0, d0, h2, es, ed, mg, rr, ss, bar, wg, bvec, scale, o_ref):
    o16 = _e_common(n0, d0, h2, es, ed, mg, rr, ss, bar)
    h1 = (jnp.dot(o16, wg[...], preferred_element_type=_f32) * scale[...]
          + bvec[...])
    o_ref[...] = jnp.concatenate([h1, jnp.zeros_like(h1)], axis=1)


def _row_spec(w):
    return pl.BlockSpec((BN, w), lambda i: (i, 0))


def _full_spec(h, w):
    return pl.BlockSpec((h, w), lambda i: (0, 0))


_a0_call = pl.pallas_call(
    _a0_body,
    grid=(GRID,),
    in_specs=[_row_spec(DD), _full_spec(DD, DD), _row_spec(1)],
    out_specs=_row_spec(2 * DD),
    out_shape=jax.ShapeDtypeStruct((NN, 2 * DD), _f32),
)

_c_call = pl.pallas_call(
    _c_body,
    grid=(GRID,),
    in_specs=[_row_spec(DD), _row_spec(2 * DD), _row_spec(1),
              _full_spec(1, DD), _full_spec(DD, DD),
              _full_spec(DD, HH), _full_spec(DD, HH)],
    out_specs=[_row_spec(2 * DD), _row_spec(HH), _row_spec(HH),
               pl.BlockSpec((1, 1, HH), lambda i: (i, 0, 0))],
    out_shape=[jax.ShapeDtypeStruct((NN, 2 * DD), _f32),
               jax.ShapeDtypeStruct((NN, HH), _f32),
               jax.ShapeDtypeStruct((NN, HH), _f32),
               jax.ShapeDtypeStruct((GRID, 1, HH), _f32)],
)

_e_call = pl.pallas_call(
    _e_body,
    grid=(GRID,),
    in_specs=[_row_spec(DD), _row_spec(16),
              _row_spec(2 * DD), _row_spec(HH), _row_spec(HH),
              _full_spec(1, HH), _full_spec(HH, DD), _full_spec(DD, CC),
              _full_spec(1, CC), _full_spec(CC, DD), _full_spec(1, DD),
              _row_spec(1)],
    out_specs=_row_spec(2 * DD),
    out_shape=jax.ShapeDtypeStruct((NN, 2 * DD), _f32),
)


def kernel(x, edge_index, Wg0, Wg12, bg, Wa, a_src, a_dst, ba, Wout, bout):
    src = edge_index[0]
    dst = edge_index[1]

    sc_deg, sc_edge = _sc_kernels()
    degp = sc_deg(dst)
    deg = degp.reshape(NPACK * 8, 16)[:NN, 0] + 1.0
    dinv = lax.rsqrt(deg)[:, None]

    eye8 = jnp.eye(HH, dtype=_f32)
    rr = jnp.repeat(eye8, CC, axis=1)
    ss = jnp.tile(jnp.eye(CC, dtype=_f32), (HH, 1)) / HH

    z1d = jnp.zeros((1, DD), _f32)
    zdd = jnp.zeros((DD, DD), _f32)
    zdh = jnp.zeros((DD, HH), _f32)
    z1h = jnp.zeros((1, CC), _f32)
    zcd = jnp.zeros((CC, DD), _f32)
    ms_l = [(a_src[i][:, :, None] * eye8[:, None, :]).reshape(DD, HH)
            for i in range(3)]
    md_l = [(a_dst[i][:, :, None] * eye8[:, None, :]).reshape(DD, HH)
            for i in range(3)]
    wout_p = jnp.pad(Wout, ((0, 0), (0, DD - NC7)))
    bout_p = jnp.pad(bout, (0, DD - NC7))[None]
    bg6 = jnp.stack([bg[0][None], z1d, bg[1][None], z1d, bg[2][None], z1d])
    wa6 = jnp.stack([Wa[0], zdd, Wa[1], zdd, Wa[2], zdd])
    ms6 = jnp.stack([ms_l[0], zdh, ms_l[1], zdh, ms_l[2], zdh])
    md6 = jnp.stack([md_l[0], zdh, md_l[1], zdh, md_l[2], zdh])
    ba6 = jnp.stack([z1h, ba[0][None], z1h, ba[1][None], z1h, ba[2][None]])
    wn6 = jnp.stack([zcd, Wg12[0], zcd, Wg12[1], zcd, wout_p])
    bo6 = jnp.stack([z1d, z1d, z1d, z1d, z1d, bout_p])
    parity6 = jnp.arange(6, dtype=_i32) % 2
    last6 = jnp.arange(6, dtype=_i32) == 5

    vmz = jnp.zeros((NN, DD), _f32)
    hs0 = _a0_call(x, Wg0, dinv)
    init = (hs0, vmz, jnp.zeros((NN, HH), _f32), jnp.zeros((NN, HH), _f32),
            jnp.zeros((1, HH), _f32))

    def body(carry, xs):
        hs, vm, es, ed, mg = carry
        bgi, wai, msi, mdi, bai, wni, boi, par, lastf = xs
        nump, denp = sc_edge(hs, vm, src, dst)

        def c_branch(_):
            hs2, es2, ed2, mparts = _c_call(nump, hs, dinv, bgi, wai,
                                            msi, mdi)
            mg2 = jnp.max(mparts[:, 0, :], axis=0, keepdims=True)
            mprime = jnp.maximum(0.0, mg2 + ed2)
            vm2 = jnp.concatenate(
                [ed2, ed2, mprime, mprime,
                 jnp.zeros((NN, DD - 4 * HH), _f32)], axis=1)
            return hs2, vm2, es2, ed2, mg2

        def e_branch(_):
            sc = jnp.where(lastf, jnp.ones_like(dinv), dinv)
            den16 = denp.reshape(NPACK * 8, 16)[:NN]
            hs2 = _e_call(nump, den16, hs, es, ed, mg, rr, ss, bai, wni,
                          boi, sc)
            return hs2, vmz, es, ed, mg

        return lax.cond(par == 0, c_branch, e_branch, 0), 0.0

    (hsf, _, _, _, _), _ = lax.scan(
        body, init, (bg6, wa6, ms6, md6, ba6, wn6, bo6, parity6, last6))
    return hsf[:, :NC7]


# speedup vs baseline: 9.0659x; 1.0398x over previous
import functools

import jax
import jax.numpy as jnp
from jax import lax
from jax.experimental import pallas as pl
from jax.experimental.pallas import tpu as pltpu
from jax.experimental.pallas import tpu_sc as plsc

NN = 10000
NC7 = 7
EE = 320000
DD = 128
HH = 8
CC = 16
NCORE = 1
NSUB = 16
NWORK = NCORE * NSUB
EPW = EE // NWORK
K = 32
NCHUNK = EPW // K
KD = 16
NCHUNKD = EPW // KD
CH = 80
ZU = 16
NZN = NN // ZU
NPACK = 1280

_f32 = jnp.float32
_i32 = jnp.int32

_GDN = lax.GatherDimensionNumbers(
    offset_dims=(), collapsed_slice_dims=(0,), start_index_map=(0,))


def _lane_bcast(v, h):
    idx = jnp.full((16, 1), h, _i32)
    return lax.gather(v, idx, _GDN, (1,),
                      mode=lax.GatherScatterMode.PROMISE_IN_BOUNDS)


def _zero_rows(zb, width, nrows):
    def body(i, carry):
        for j in range(width // 16):
            zb[i, pl.ds(j * 16, 16)] = jnp.zeros((16,), _f32)
        return carry
    lax.fori_loop(0, nrows, body, 0)


def _strided_units(s, nunits, fn):
    nper = -(-nunits // NSUB)

    def body(t, carry):
        cid = s + NSUB * t
        @pl.when(cid < nunits)
        def _():
            fn(cid)
        return carry
    lax.fori_loop(0, nper, body, 0)


def _wid(c, s):
    return c * NSUB + s


def _sc_deg_body(dst_hbm, out_hbm, didx, dpack, ones, acc):
    c = lax.axis_index("c")
    s = lax.axis_index("s")
    _zero_rows(ones, DD, KD)

    lane = lax.iota(_i32, 16)
    one_row = jnp.where(lane == 0, 1.0, 0.0).astype(_f32)
    zrow = jnp.zeros((16,), _f32)

    _strided_units(s, NPACK // ZU, lambda u: pltpu.sync_copy(
        ones.at[pl.ds(0, ZU)], acc.at[pl.ds(u * ZU, ZU)]))
    plsc.subcore_barrier()

    base0 = _wid(c, s) * EPW

    def chunk(g, carry):
        pltpu.sync_copy(dst_hbm.at[pl.ds(base0 + g * KD, KD)], didx)

        def grp(v, carry2):
            dv = didx[pl.ds(v * 16, 16)]
            dpack[pl.ds(v * 16, 16)] = lax.shift_right_logical(dv, 3)
            offv = (dv & 7) * 16
            for l in range(16):
                e = v * 16 + l
                off = offv[l]
                for j in range(DD // 16):
                    ones[e, pl.ds(j * 16, 16)] = zrow
                ones[e, pl.ds(off, 16)] = one_row
            return carry2
        lax.fori_loop(0, KD // 16, grp, 0)

        pltpu.sync_copy(ones, acc.at[dpack], add=True)
        return carry
    lax.fori_loop(0, NCHUNKD, chunk, 0)

    plsc.subcore_barrier()
    pltpu.sync_copy(acc.at[pl.ds(s * CH, CH)], out_hbm.at[pl.ds(s * CH, CH)])


def _sc_edge_body(hs_hbm, vm_hbm, src_hbm, dst_hbm, num_hbm, den_hbm,
                  sidx0, didx0, dpack0, sidx1, didx1, dpack1,
                  hsbuf, vmbuf, sbuf, exbuf, accn, accd, semg, semd):
    c = lax.axis_index("c")
    s = lax.axis_index("s")
    _zero_rows(sbuf, DD, K)

    _strided_units(s, NZN, lambda u: pltpu.sync_copy(
        sbuf.at[pl.ds(0, ZU)], accn.at[pl.ds(u * ZU, ZU)]))
    _strided_units(s, NPACK // ZU, lambda u: pltpu.sync_copy(
        sbuf.at[pl.ds(0, ZU)], accd.at[pl.ds(u * ZU, ZU)]))
    plsc.subcore_barrier()

    base0 = _wid(c, s) * EPW
    zrow = jnp.zeros((16,), _f32)
    bufs = ((sidx0, didx0, dpack0), (sidx1, didx1, dpack1))

    def _drain(b):
        pltpu.make_async_copy(sbuf, accn.at[bufs[b][1]], semd).wait()
        pltpu.make_async_copy(exbuf, accd.at[bufs[b][2]], semd).wait()

    def step(bufset, g, drain):
        sidx, didx, dpack = bufset
        base = base0 + g * K
        pltpu.sync_copy(src_hbm.at[pl.ds(base, K)], sidx)
        pltpu.sync_copy(dst_hbm.at[pl.ds(base, K)], didx)
        cp_hs = pltpu.async_copy(hs_hbm.at[sidx], hsbuf, semg)
        cp_vm = pltpu.async_copy(vm_hbm.at[didx], vmbuf, semg)
        drain()
        cp_hs.wait()
        cp_vm.wait()

        def grp(v, carry2):
            dv = didx[pl.ds(v * 16, 16)]
            dpack[pl.ds(v * 16, 16)] = lax.shift_right_logical(dv, 3)
            offv = (dv & 7) * 16
            for l in range(16):
                e = v * 16 + l
                off = offv[l]
                u = hsbuf[e, pl.ds(DD, 16)]
                edv = vmbuf[e, pl.ds(0, 16)]
                mm = vmbuf[e, pl.ds(16, 16)]
                z = u + edv
                lr = jnp.where(z > 0, z, 0.2 * z)
                ex = jnp.exp(lr - mm)
                for j in range(DD // 16):
                    exbuf[e, pl.ds(j * 16, 16)] = zrow
                exbuf[e, pl.ds(off, 16)] = ex
                for h in range(HH):
                    bb = _lane_bcast(ex, h)
                    sbuf[e, pl.ds(h * 16, 16)] = (
                        hsbuf[e, pl.ds(h * 16, 16)] * bb)
            return carry2
        lax.fori_loop(0, K // 16, grp, 0)

        pltpu.async_copy(sbuf, accn.at[didx], semd, add=True)
        pltpu.async_copy(exbuf, accd.at[dpack], semd, add=True)

    def chunk2(gg, carry):
        def drain0():
            @pl.when(gg > 0)
            def _():
                _drain(1)
        step(bufs[0], gg * 2, drain0)
        step(bufs[1], gg * 2 + 1, lambda: _drain(0))
        return carry
    lax.fori_loop(0, NCHUNK // 2, chunk2, 0)
    step(bufs[0], NCHUNK - 1, lambda: _drain(1))
    _drain(0)

    plsc.subcore_barrier()
    _strided_units(s, NZN, lambda u: pltpu.sync_copy(
        accn.at[pl.ds(u * ZU, ZU)], num_hbm.at[pl.ds(u * ZU, ZU)]))
    pltpu.sync_copy(accd.at[pl.ds(s * CH, CH)],
                    den_hbm.at[pl.ds(s * CH, CH)])


@functools.cache
def _sc_kernels():
    mesh = plsc.VectorSubcoreMesh(
        core_axis_name="c", subcore_axis_name="s",
        num_cores=NCORE, num_subcores=NSUB)
    deg = pl.kernel(
        _sc_deg_body,
        out_type=jax.ShapeDtypeStruct((NPACK, DD), _f32),
        mesh=mesh,
        scratch_types=[
            pltpu.VMEM((KD,), _i32),
            pltpu.VMEM((KD,), _i32),
            pltpu.VMEM((KD, DD), _f32),
            pltpu.VMEM_SHARED((NPACK, DD), _f32),
        ])
    edge = pl.kernel(
        _sc_edge_body,
        out_type=[jax.ShapeDtypeStruct((NN, DD), _f32),
                  jax.ShapeDtypeStruct((NPACK, DD), _f32)],
        mesh=mesh,
        scratch_types=[
            pltpu.VMEM((K,), _i32),
            pltpu.VMEM((K,), _i32),
            pltpu.VMEM((K,), _i32),
            pltpu.VMEM((K,), _i32),
            pltpu.VMEM((K,), _i32),
            pltpu.VMEM((K,), _i32),
            pltpu.VMEM((K, 2 * DD), _f32),
            pltpu.VMEM((K, DD), _f32),
            pltpu.VMEM((K, DD), _f32),
            pltpu.VMEM((K, DD), _f32),
            pltpu.VMEM_SHARED((NN, DD), _f32),
            pltpu.VMEM_SHARED((NPACK, DD), _f32),
            pltpu.SemaphoreType.DMA,
            pltpu.SemaphoreType.DMA,
        ])
    return deg, edge


BN = 1000
GRID = NN // BN


def _a0_body(x_ref, w_ref, dinv_ref, o_ref):
    h1 = jnp.dot(x_ref[...], w_ref[...],
                 preferred_element_type=_f32) * dinv_ref[...]
    o_ref[...] = jnp.concatenate([h1, jnp.zeros_like(h1)], axis=1)


def _c_body(p0, h1p, dinv, bgr, wa, ms, md, hso, eso, edo, mo):
    hg = jnp.maximum(
        (p0[...] + h1p[...][:, :DD]) * dinv[...] + bgr[...], 0.0)
    h2 = jnp.dot(hg, wa[...], preferred_element_type=_f32)
    es = jnp.dot(h2, ms[...], preferred_element_type=_f32)
    ed = jnp.dot(h2, md[...], preferred_element_type=_f32)
    hso[...] = jnp.concatenate(
        [h2, es, es, jnp.zeros((h2.shape[0], 2 * DD - DD - 2 * HH), _f32)],
        axis=1)
    eso[...] = es
    edo[...] = ed
    mo[...] = jnp.max(es, axis=0, keepdims=True)[None]


def _e_common(n0, d0, hs, es, ed, mg, rr, ss, bar):
    esv = es[...]
    edv = ed[...]
    z = esv + edv
    lr = jnp.where(z > 0, z, 0.2 * z)
    mp = jnp.maximum(0.0, mg[...] + edv)
    exs = jnp.exp(lr - mp)
    den = d0[...][:, :HH] + exs
    num = n0[...] + jnp.dot(exs, rr[...],
                            preferred_element_type=_f32) * hs[...][:, :DD]
    denw = jnp.dot(den, rr[...], preferred_element_type=_f32) + 1e-16
    return jnp.dot(num / denw, ss[...], preferred_element_type=_f32) + bar[...]


def _e_body(n0, d0, h2, es, ed, mg, rr, ss, bar, wg, bvec, scale, o_ref):
    o16 = _e_common(n0, d0, h2, es, ed, mg, rr, ss, bar)
    h1 = (jnp.dot(o16, wg[...], preferred_element_type=_f32) * scale[...]
          + bvec[...])
    o_ref[...] = jnp.concatenate([h1, jnp.zeros_like(h1)], axis=1)


def _row_spec(w):
    return pl.BlockSpec((BN, w), lambda i: (i, 0))


def _full_spec(h, w):
    return pl.BlockSpec((h, w), lambda i: (0, 0))


_a0_call = pl.pallas_call(
    _a0_body,
    grid=(GRID,),
    in_specs=[_row_spec(DD), _full_spec(DD, DD), _row_spec(1)],
    out_specs=_row_spec(2 * DD),
    out_shape=jax.ShapeDtypeStruct((NN, 2 * DD), _f32),
)

_c_call = pl.pallas_call(
    _c_body,
    grid=(GRID,),
    in_specs=[_row_spec(DD), _row_spec(2 * DD), _row_spec(1),
              _full_spec(1, DD), _full_spec(DD, DD),
              _full_spec(DD, HH), _full_spec(DD, HH)],
    out_specs=[_row_spec(2 * DD), _row_spec(HH), _row_spec(HH),
               pl.BlockSpec((1, 1, HH), lambda i: (i, 0, 0))],
    out_shape=[jax.ShapeDtypeStruct((NN, 2 * DD), _f32),
               jax.ShapeDtypeStruct((NN, HH), _f32),
               jax.ShapeDtypeStruct((NN, HH), _f32),
               jax.ShapeDtypeStruct((GRID, 1, HH), _f32)],
)

_e_call = pl.pallas_call(
    _e_body,
    grid=(GRID,),
    in_specs=[_row_spec(DD), _row_spec(16),
              _row_spec(2 * DD), _row_spec(HH), _row_spec(HH),
              _full_spec(1, HH), _full_spec(HH, DD), _full_spec(DD, CC),
              _full_spec(1, CC), _full_spec(CC, DD), _full_spec(1, DD),
              _row_spec(1)],
    out_specs=_row_spec(2 * DD),
    out_shape=jax.ShapeDtypeStruct((NN, 2 * DD), _f32),
)


def kernel(x, edge_index, Wg0, Wg12, bg, Wa, a_src, a_dst, ba, Wout, bout):
    src = edge_index[0]
    dst = edge_index[1]

    sc_deg, sc_edge = _sc_kernels()
    degp = sc_deg(dst)
    deg = degp.reshape(NPACK * 8, 16)[:NN, 0] + 1.0
    dinv = lax.rsqrt(deg)[:, None]

    eye8 = jnp.eye(HH, dtype=_f32)
    rr = jnp.repeat(eye8, CC, axis=1)
    ss = jnp.tile(jnp.eye(CC, dtype=_f32), (HH, 1)) / HH

    z1d = jnp.zeros((1, DD), _f32)
    zdd = jnp.zeros((DD, DD), _f32)
    zdh = jnp.zeros((DD, HH), _f32)
    z1h = jnp.zeros((1, CC), _f32)
    zcd = jnp.zeros((CC, DD), _f32)
    ms_l = [(a_src[i][:, :, None] * eye8[:, None, :]).reshape(DD, HH)
            for i in range(3)]
    md_l = [(a_dst[i][:, :, None] * eye8[:, None, :]).reshape(DD, HH)
            for i in range(3)]
    wout_p = jnp.pad(Wout, ((0, 0), (0, DD - NC7)))
    bout_p = jnp.pad(bout, (0, DD - NC7))[None]
    bg6 = jnp.stack([bg[0][None], z1d, bg[1][None], z1d, bg[2][None], z1d])
    wa6 = jnp.stack([Wa[0], zdd, Wa[1], zdd, Wa[2], zdd])
    ms6 = jnp.stack([ms_l[0], zdh, ms_l[1], zdh, ms_l[2], zdh])
    md6 = jnp.stack([md_l[0], zdh, md_l[1], zdh, md_l[2], zdh])
    ba6 = jnp.stack([z1h, ba[0][None], z1h, ba[1][None], z1h, ba[2][None]])
    wn6 = jnp.stack([zcd, Wg12[0], zcd, Wg12[1], zcd, wout_p])
    bo6 = jnp.stack([z1d, z1d, z1d, z1d, z1d, bout_p])
    parity6 = jnp.arange(6, dtype=_i32) % 2
    last6 = jnp.arange(6, dtype=_i32) == 5

    vmz = jnp.zeros((NN, DD), _f32)
    hs0 = _a0_call(x, Wg0, dinv)
    init = (hs0, vmz, jnp.zeros((NN, HH), _f32), jnp.zeros((NN, HH), _f32),
            jnp.zeros((1, HH), _f32))

    def body(carry, xs):
        hs, vm, es, ed, mg = carry
        bgi, wai, msi, mdi, bai, wni, boi, par, lastf = xs
        nump, denp = sc_edge(hs, vm, src, dst)

        def c_branch(_):
            hs2, es2, ed2, mparts = _c_call(nump, hs, dinv, bgi, wai,
                                            msi, mdi)
            mg2 = jnp.max(mparts[:, 0, :], axis=0, keepdims=True)
            mprime = jnp.maximum(0.0, mg2 + ed2)
            vm2 = jnp.concatenate(
                [ed2, ed2, mprime, mprime,
                 jnp.zeros((NN, DD - 4 * HH), _f32)], axis=1)
            return hs2, vm2, es2, ed2, mg2

        def e_branch(_):
            sc = jnp.where(lastf, jnp.ones_like(dinv), dinv)
            den16 = denp.reshape(NPACK * 8, 16)[:NN]
            hs2 = _e_call(nump, den16, hs, es, ed, mg, rr, ss, bai, wni,
                          boi, sc)
            return hs2, vmz, es, ed, mg

        return lax.cond(par == 0, c_branch, e_branch, 0), 0.0

    (hsf, _, _, _, _), _ = lax.scan(
        body, init, (bg6, wa6, ms6, md6, ba6, wn6, bo6, parity6, last6))
    return hsf[:, :NC7]


# speedup vs baseline: 9.9827x; 1.1011x over previous
import functools

import jax
import jax.numpy as jnp
from jax import lax
from jax.experimental import pallas as pl
from jax.experimental.pallas import tpu as pltpu
from jax.experimental.pallas import tpu_sc as plsc

NN = 10000
NC7 = 7
EE = 320000
DD = 128
HH = 8
CC = 16
NCORE = 1
NSUB = 16
NWORK = NCORE * NSUB
EPW = EE // NWORK
K = 16
NCHUNK = EPW // K
KD = 16
NCHUNKD = EPW // KD
CH = 80
ZU = 16
NZN = NN // ZU
NPACK = 1280

_f32 = jnp.float32
_i32 = jnp.int32

_GDN = lax.GatherDimensionNumbers(
    offset_dims=(), collapsed_slice_dims=(0,), start_index_map=(0,))


def _lane_bcast(v, h):
    idx = jnp.full((16, 1), h, _i32)
    return lax.gather(v, idx, _GDN, (1,),
                      mode=lax.GatherScatterMode.PROMISE_IN_BOUNDS)


def _zero_rows(zb, width, nrows):
    def body(i, carry):
        for j in range(width // 16):
            zb[i, pl.ds(j * 16, 16)] = jnp.zeros((16,), _f32)
        return carry
    lax.fori_loop(0, nrows, body, 0)


def _strided_units(s, nunits, fn):
    nper = -(-nunits // NSUB)

    def body(t, carry):
        cid = s + NSUB * t
        @pl.when(cid < nunits)
        def _():
            fn(cid)
        return carry
    lax.fori_loop(0, nper, body, 0)


def _wid(c, s):
    return c * NSUB + s


def _sc_deg_body(dst_hbm, out_hbm, didx, dpack, ones, acc):
    c = lax.axis_index("c")
    s = lax.axis_index("s")
    _zero_rows(ones, DD, KD)

    lane = lax.iota(_i32, 16)
    one_row = jnp.where(lane == 0, 1.0, 0.0).astype(_f32)
    zrow = jnp.zeros((16,), _f32)

    _strided_units(s, NPACK // ZU, lambda u: pltpu.sync_copy(
        ones.at[pl.ds(0, ZU)], acc.at[pl.ds(u * ZU, ZU)]))
    plsc.subcore_barrier()

    base0 = _wid(c, s) * EPW

    def chunk(g, carry):
        pltpu.sync_copy(dst_hbm.at[pl.ds(base0 + g * KD, KD)], didx)

        def grp(v, carry2):
            dv = didx[pl.ds(v * 16, 16)]
            dpack[pl.ds(v * 16, 16)] = lax.shift_right_logical(dv, 3)
            offv = (dv & 7) * 16
            for l in range(16):
                e = v * 16 + l
                off = offv[l]
                for j in range(DD // 16):
                    ones[e, pl.ds(j * 16, 16)] = zrow
                ones[e, pl.ds(off, 16)] = one_row
            return carry2
        lax.fori_loop(0, KD // 16, grp, 0)

        pltpu.sync_copy(ones, acc.at[dpack], add=True)
        return carry
    lax.fori_loop(0, NCHUNKD, chunk, 0)

    plsc.subcore_barrier()
    pltpu.sync_copy(acc.at[pl.ds(s * CH, CH)], out_hbm.at[pl.ds(s * CH, CH)])


def _sc_edge_body(hs_hbm, vm_hbm, src_hbm, dst_hbm, num_hbm, den_hbm,
                  sidx0, didx0, dpack0, sidx1, didx1, dpack1,
                  hsbuf0, hsbuf1, vmbuf0, vmbuf1, sbuf, exbuf,
                  accn, accd, semg0, semg1, semd):
    c = lax.axis_index("c")
    s = lax.axis_index("s")
    _zero_rows(sbuf, DD, K)

    _strided_units(s, NZN, lambda u: pltpu.sync_copy(
        sbuf.at[pl.ds(0, ZU)], accn.at[pl.ds(u * ZU, ZU)]))
    _strided_units(s, NPACK // ZU, lambda u: pltpu.sync_copy(
        sbuf.at[pl.ds(0, ZU)], accd.at[pl.ds(u * ZU, ZU)]))
    plsc.subcore_barrier()

    base0 = _wid(c, s) * EPW
    zrow = jnp.zeros((16,), _f32)
    sets = ((sidx0, didx0, dpack0, hsbuf0, vmbuf0, semg0),
            (sidx1, didx1, dpack1, hsbuf1, vmbuf1, semg1))

    def fire(b, g):
        sidx, didx, _, hsb, vmb, sg = sets[b]
        base = base0 + g * K
        pltpu.sync_copy(src_hbm.at[pl.ds(base, K)], sidx)
        pltpu.sync_copy(dst_hbm.at[pl.ds(base, K)], didx)
        pltpu.async_copy(hs_hbm.at[sidx], hsb, sg)
        pltpu.async_copy(vm_hbm.at[didx], vmb, sg)

    def wait_g(b):
        sidx, didx, _, hsb, vmb, sg = sets[b]
        pltpu.make_async_copy(hs_hbm.at[sidx], hsb, sg).wait()
        pltpu.make_async_copy(vm_hbm.at[didx], vmb, sg).wait()

    def drain_s(b):
        _, didx, dpack, _, _, _ = sets[b]
        pltpu.make_async_copy(sbuf, accn.at[didx], semd).wait()
        pltpu.make_async_copy(exbuf, accd.at[dpack], semd).wait()

    def compute_scatter(b):
        _, didx, dpack, hsb, vmb, _ = sets[b]
        dv = didx[pl.ds(0, 16)]
        dpack[pl.ds(0, 16)] = lax.shift_right_logical(dv, 3)
        offv = (dv & 7) * 16
        for e in range(16):
            off = offv[e]
            u = hsb[e, pl.ds(DD, 16)]
            edv = vmb[e, pl.ds(0, 16)]
            mm = vmb[e, pl.ds(16, 16)]
            z = u + edv
            lr = jnp.where(z > 0, z, 0.2 * z)
            ex = jnp.exp(lr - mm)
            for j in range(DD // 16):
                exbuf[e, pl.ds(j * 16, 16)] = zrow
            exbuf[e, pl.ds(off, 16)] = ex
            for h in range(HH):
                bb = _lane_bcast(ex, h)
                sbuf[e, pl.ds(h * 16, 16)] = hsb[e, pl.ds(h * 16, 16)] * bb
        pltpu.async_copy(sbuf, accn.at[didx], semd, add=True)
        pltpu.async_copy(exbuf, accd.at[dpack], semd, add=True)

    fire(0, 0)

    def body2(gg, carry):
        g0 = gg * 2
        wait_g(0)
        @pl.when(gg > 0)
        def _():
            drain_s(1)
        fire(1, g0 + 1)
        compute_scatter(0)
        wait_g(1)
        drain_s(0)
        @pl.when(gg < NCHUNK // 2 - 1)
        def _():
            fire(0, g0 + 2)
        compute_scatter(1)
        return carry
    lax.fori_loop(0, NCHUNK // 2, body2, 0)
    drain_s(1)

    plsc.subcore_barrier()
    _strided_units(s, NZN, lambda u: pltpu.sync_copy(
        accn.at[pl.ds(u * ZU, ZU)], num_hbm.at[pl.ds(u * ZU, ZU)]))
    pltpu.sync_copy(accd.at[pl.ds(s * CH, CH)],
                    den_hbm.at[pl.ds(s * CH, CH)])


@functools.cache
def _sc_kernels():
    mesh = plsc.VectorSubcoreMesh(
        core_axis_name="c", subcore_axis_name="s",
        num_cores=NCORE, num_subcores=NSUB)
    deg = pl.kernel(
        _sc_deg_body,
        out_type=jax.ShapeDtypeStruct((NPACK, DD), _f32),
        mesh=mesh,
        scratch_types=[
            pltpu.VMEM((KD,), _i32),
            pltpu.VMEM((KD,), _i32),
            pltpu.VMEM((KD, DD), _f32),
            pltpu.VMEM_SHARED((NPACK, DD), _f32),
        ])
    edge = pl.kernel(
        _sc_edge_body,
        out_type=[jax.ShapeDtypeStruct((NN, DD), _f32),
                  jax.ShapeDtypeStruct((NPACK, DD), _f32)],
        mesh=mesh,
        scratch_types=[
            pltpu.VMEM((K,), _i32),
            pltpu.VMEM((K,), _i32),
            pltpu.VMEM((K,), _i32),
            pltpu.VMEM((K,), _i32),
            pltpu.VMEM((K,), _i32),
            pltpu.VMEM((K,), _i32),
            pltpu.VMEM((K, 2 * DD), _f32),
            pltpu.VMEM((K, 2 * DD), _f32),
            pltpu.VMEM((K, DD), _f32),
            pltpu.VMEM((K, DD), _f32),
            pltpu.VMEM((K, DD), _f32),
            pltpu.VMEM((K, DD), _f32),
            pltpu.VMEM_SHARED((NN, DD), _f32),
            pltpu.VMEM_SHARED((NPACK, DD), _f32),
            pltpu.SemaphoreType.DMA,
            pltpu.SemaphoreType.DMA,
            pltpu.SemaphoreType.DMA,
        ])
    return deg, edge


BN = 1000
GRID = NN // BN


def _a0_body(x_ref, w_ref, dinv_ref, o_ref):
    h1 = jnp.dot(x_ref[...], w_ref[...],
                 preferred_element_type=_f32) * dinv_ref[...]
    o_ref[...] = jnp.concatenate([h1, jnp.zeros_like(h1)], axis=1)


def _c_body(p0, h1p, dinv, bgr, wa, ms, md, hso, eso, edo, mo):
    hg = jnp.maximum(
        (p0[...] + h1p[...][:, :DD]) * dinv[...] + bgr[...], 0.0)
    h2 = jnp.dot(hg, wa[...], preferred_element_type=_f32)
    es = jnp.dot(h2, ms[...], preferred_element_type=_f32)
    ed = jnp.dot(h2, md[...], preferred_element_type=_f32)
    hso[...] = jnp.concatenate(
        [h2, es, es, jnp.zeros((h2.shape[0], 2 * DD - DD - 2 * HH), _f32)],
        axis=1)
    eso[...] = es
    edo[...] = ed
    mo[...] = jnp.max(es, axis=0, keepdims=True)[None]


def _e_common(n0, d0, hs, es, ed, mg, rr, ss, bar):
    esv = es[...]
    edv = ed[...]
    z = esv + edv
    lr = jnp.where(z > 0, z, 0.2 * z)
    mp = jnp.maximum(0.0, mg[...] + edv)
    exs = jnp.exp(lr - mp)
    den = d0[...][:, :HH] + exs
    num = n0[...] + jnp.dot(exs, rr[...],
                            preferred_element_type=_f32) * hs[...][:, :DD]
    denw = jnp.dot(den, rr[...], preferred_element_type=_f32) + 1e-16
    return jnp.dot(num / denw, ss[...], preferred_element_type=_f32) + bar[...]


def _e_body(n0, d0, h2, es, ed, mg, rr, ss, bar, wg, bvec, scale, o_ref):
    o16 = _e_common(n0, d0, h2, es, ed, mg, rr, ss, bar)
    h1 = (jnp.dot(o16, wg[...], preferred_element_type=_f32) * scale[...]
          + bvec[...])
    o_ref[...] = jnp.concatenate([h1, jnp.zeros_like(h1)], axis=1)


def _row_spec(w):
    return pl.BlockSpec((BN, w), lambda i: (i, 0))


def _full_spec(h, w):
    return pl.BlockSpec((h, w), lambda i: (0, 0))


_a0_call = pl.pallas_call(
    _a0_body,
    grid=(GRID,),
    in_specs=[_row_spec(DD), _full_spec(DD, DD), _row_spec(1)],
    out_specs=_row_spec(2 * DD),
    out_shape=jax.ShapeDtypeStruct((NN, 2 * DD), _f32),
)

_c_call = pl.pallas_call(
    _c_body,
    grid=(GRID,),
    in_specs=[_row_spec(DD), _row_spec(2 * DD), _row_spec(1),
              _full_spec(1, DD), _full_spec(DD, DD),
              _full_spec(DD, HH), _full_spec(DD, HH)],
    out_specs=[_row_spec(2 * DD), _row_spec(HH), _row_spec(HH),
               pl.BlockSpec((1, 1, HH), lambda i: (i, 0, 0))],
    out_shape=[jax.ShapeDtypeStruct((NN, 2 * DD), _f32),
               jax.ShapeDtypeStruct((NN, HH), _f32),
               jax.ShapeDtypeStruct((NN, HH), _f32),
               jax.ShapeDtypeStruct((GRID, 1, HH), _f32)],
)

_e_call = pl.pallas_call(
    _e_body,
    grid=(GRID,),
    in_specs=[_row_spec(DD), _row_spec(16),
              _row_spec(2 * DD), _row_spec(HH), _row_spec(HH),
              _full_spec(1, HH), _full_spec(HH, DD), _full_spec(DD, CC),
              _full_spec(1, CC), _full_spec(CC, DD), _full_spec(1, DD),
              _row_spec(1)],
    out_specs=_row_spec(2 * DD),
    out_shape=jax.ShapeDtypeStruct((NN, 2 * DD), _f32),
)


def kernel(x, edge_index, Wg0, Wg12, bg, Wa, a_src, a_dst, ba, Wout, bout):
    src = edge_index[0]
    dst = edge_index[1]

    sc_deg, sc_edge = _sc_kernels()
    degp = sc_deg(dst)
    deg = degp.reshape(NPACK * 8, 16)[:NN, 0] + 1.0
    dinv = lax.rsqrt(deg)[:, None]

    eye8 = jnp.eye(HH, dtype=_f32)
    rr = jnp.repeat(eye8, CC, axis=1)
    ss = jnp.tile(jnp.eye(CC, dtype=_f32), (HH, 1)) / HH

    z1d = jnp.zeros((1, DD), _f32)
    zdd = jnp.zeros((DD, DD), _f32)
    zdh = jnp.zeros((DD, HH), _f32)
    z1h = jnp.zeros((1, CC), _f32)
    zcd = jnp.zeros((CC, DD), _f32)
    ms_l = [(a_src[i][:, :, None] * eye8[:, None, :]).reshape(DD, HH)
            for i in range(3)]
    md_l = [(a_dst[i][:, :, None] * eye8[:, None, :]).reshape(DD, HH)
            for i in range(3)]
    wout_p = jnp.pad(Wout, ((0, 0), (0, DD - NC7)))
    bout_p = jnp.pad(bout, (0, DD - NC7))[None]
    bg6 = jnp.stack([bg[0][None], z1d, bg[1][None], z1d, bg[2][None], z1d])
    wa6 = jnp.stack([Wa[0], zdd, Wa[1], zdd, Wa[2], zdd])
    ms6 = jnp.stack([ms_l[0], zdh, ms_l[1], zdh, ms_l[2], zdh])
    md6 = jnp.stack([md_l[0], zdh, md_l[1], zdh, md_l[2], zdh])
    ba6 = jnp.stack([z1h, ba[0][None], z1h, ba[1][None], z1h, ba[2][None]])
    wn6 = jnp.stack([zcd, Wg12[0], zcd, Wg12[1], zcd, wout_p])
    bo6 = jnp.stack([z1d, z1d, z1d, z1d, z1d, bout_p])
    parity6 = jnp.arange(6, dtype=_i32) % 2
    last6 = jnp.arange(6, dtype=_i32) == 5

    vmz = jnp.zeros((NN, DD), _f32)
    hs0 = _a0_call(x, Wg0, dinv)
    init = (hs0, vmz, jnp.zeros((NN, HH), _f32), jnp.zeros((NN, HH), _f32),
            jnp.zeros((1, HH), _f32))

    def body(carry, xs):
        hs, vm, es, ed, mg = carry
        bgi, wai, msi, mdi, bai, wni, boi, par, lastf = xs
        nump, denp = sc_edge(hs, vm, src, dst)

        def c_branch(_):
            hs2, es2, ed2, mparts = _c_call(nump, hs, dinv, bgi, wai,
                                            msi, mdi)
            mg2 = jnp.max(mparts[:, 0, :], axis=0, keepdims=True)
            mprime = jnp.maximum(0.0, mg2 + ed2)
            vm2 = jnp.concatenate(
                [ed2, ed2, mprime, mprime,
                 jnp.zeros((NN, DD - 4 * HH), _f32)], axis=1)
            return hs2, vm2, es2, ed2, mg2

        def e_branch(_):
            sc = jnp.where(lastf, jnp.ones_like(dinv), dinv)
            den16 = denp.reshape(NPACK * 8, 16)[:NN]
            hs2 = _e_call(nump, den16, hs, es, ed, mg, rr, ss, bai, wni,
                          boi, sc)
            return hs2, vmz, es, ed, mg

        return lax.cond(par == 0, c_branch, e_branch, 0), 0.0

    (hsf, _, _, _, _), _ = lax.scan(
        body, init, (bg6, wa6, ms6, md6, ba6, wn6, bo6, parity6, last6))
    return hsf[:, :NC7]


# speedup vs baseline: 13.3716x; 1.3395x over previous
import functools

import jax
import jax.numpy as jnp
from jax import lax
from jax.experimental import pallas as pl
from jax.experimental.pallas import tpu as pltpu
from jax.experimental.pallas import tpu_sc as plsc

NN = 10000
NC7 = 7
EE = 320000
DD = 128
HH = 8
CC = 16
NCORE = 1
NSUB = 16
NWORK = NCORE * NSUB
EPW = EE // NWORK
K = 32
NCHUNK = EPW // K
NPACKD = 640
KD = 16
NCHUNKD = EPW // KD
CH = 80
ZU = 16
NZN = NN // ZU
NPACK = 1280

_f32 = jnp.float32
_i32 = jnp.int32

_GDN = lax.GatherDimensionNumbers(
    offset_dims=(), collapsed_slice_dims=(0,), start_index_map=(0,))


def _lane_bcast(v, h):
    idx = jnp.full((16, 1), h, _i32)
    return lax.gather(v, idx, _GDN, (1,),
                      mode=lax.GatherScatterMode.PROMISE_IN_BOUNDS)


def _zero_rows(zb, width, nrows):
    def body(i, carry):
        for j in range(width // 16):
            zb[i, pl.ds(j * 16, 16)] = jnp.zeros((16,), _f32)
        return carry
    lax.fori_loop(0, nrows, body, 0)


def _strided_units(s, nunits, fn):
    nper = -(-nunits // NSUB)

    def body(t, carry):
        cid = s + NSUB * t
        @pl.when(cid < nunits)
        def _():
            fn(cid)
        return carry
    lax.fori_loop(0, nper, body, 0)


def _wid(c, s):
    return c * NSUB + s


def _sc_deg_body(dst_hbm, out_hbm, didx, dpack, ones, acc):
    c = lax.axis_index("c")
    s = lax.axis_index("s")
    _zero_rows(ones, DD, KD)

    lane = lax.iota(_i32, 16)
    one_a = jnp.where(lane == 0, 1.0, 0.0).astype(_f32)
    one_b = jnp.where(lane == 8, 1.0, 0.0).astype(_f32)
    zrow = jnp.zeros((16,), _f32)

    _strided_units(s, NPACKD // ZU, lambda u: pltpu.sync_copy(
        ones.at[pl.ds(0, ZU)], acc.at[pl.ds(u * ZU, ZU)]))
    plsc.subcore_barrier()

    base0 = _wid(c, s) * EPW

    def chunk(g, carry):
        pltpu.sync_copy(dst_hbm.at[pl.ds(base0 + g * KD, KD)], didx)

        def grp(v, carry2):
            dv = didx[pl.ds(v * 16, 16)]
            dpack[pl.ds(v * 16, 16)] = lax.shift_right_logical(dv, 4)
            q = dv & 15
            offv = (q & 14) * 8
            pf = (q & 1).astype(_f32)
            for l in range(16):
                e = v * 16 + l
                for j in range(DD // 16):
                    ones[e, pl.ds(j * 16, 16)] = zrow
                ones[e, pl.ds(offv[l], 16)] = (one_b * pf[l]
                                               + one_a * (1.0 - pf[l]))
            return carry2
        lax.fori_loop(0, KD // 16, grp, 0)

        pltpu.sync_copy(ones, acc.at[dpack], add=True)
        return carry
    lax.fori_loop(0, NCHUNKD, chunk, 0)

    plsc.subcore_barrier()
    pltpu.sync_copy(acc.at[pl.ds(s * (NPACKD // NSUB), NPACKD // NSUB)],
                    out_hbm.at[pl.ds(s * (NPACKD // NSUB), NPACKD // NSUB)])


def _sc_edge_body(hs_hbm, vm_hbm, src_hbm, dst_hbm, num_hbm, den_hbm,
                  sidx0, didx0, dpack0, sidx1, didx1, dpack1,
                  hsbuf0, hsbuf1, vmbuf, sbuf, exbuf,
                  accn, accd, semg0, semg1, semv, semd):
    c = lax.axis_index("c")
    s = lax.axis_index("s")
    _zero_rows(sbuf, DD, K)

    _strided_units(s, NZN, lambda u: pltpu.sync_copy(
        sbuf.at[pl.ds(0, ZU)], accn.at[pl.ds(u * ZU, ZU)]))
    _strided_units(s, NPACK // ZU, lambda u: pltpu.sync_copy(
        sbuf.at[pl.ds(0, ZU)], accd.at[pl.ds(u * ZU, ZU)]))
    plsc.subcore_barrier()

    base0 = _wid(c, s) * EPW
    zrow = jnp.zeros((16,), _f32)
    sets = ((sidx0, didx0, dpack0, hsbuf0, semg0),
            (sidx1, didx1, dpack1, hsbuf1, semg1))

    def fire_hs(b, g):
        sidx, didx, _, hsb, sg = sets[b]
        base = base0 + g * K
        pltpu.sync_copy(src_hbm.at[pl.ds(base, K)], sidx)
        pltpu.sync_copy(dst_hbm.at[pl.ds(base, K)], didx)
        pltpu.async_copy(hs_hbm.at[sidx], hsb, sg)

    def fire_vm(b):
        pltpu.async_copy(vm_hbm.at[sets[b][1]], vmbuf, semv)

    def wait_g(b):
        sidx, didx, _, hsb, sg = sets[b]
        pltpu.make_async_copy(hs_hbm.at[sidx], hsb, sg).wait()
        pltpu.make_async_copy(vm_hbm.at[didx], vmbuf, semv).wait()

    def drain_s(b):
        _, didx, dpack, _, _ = sets[b]
        pltpu.make_async_copy(sbuf, accn.at[didx], semd).wait()
        pltpu.make_async_copy(exbuf, accd.at[dpack], semd).wait()

    def compute_scatter(b):
        _, didx, dpack, hsb, _ = sets[b]
        for v in range(K // 16):
            dv = didx[pl.ds(v * 16, 16)]
            dpack[pl.ds(v * 16, 16)] = lax.shift_right_logical(dv, 3)
            offv = (dv & 7) * 16
            for l in range(16):
                e = v * 16 + l
                off = offv[l]
                u = hsb[e, pl.ds(DD, 16)]
                edv = vmbuf[e, pl.ds(0, 16)]
                mm = vmbuf[e, pl.ds(16, 16)]
                z = u + edv
                lr = jnp.where(z > 0, z, 0.2 * z)
                ex = jnp.exp(lr - mm)
                for j in range(DD // 16):
                    exbuf[e, pl.ds(j * 16, 16)] = zrow
                exbuf[e, pl.ds(off, 16)] = ex
                for h in range(HH):
                    bb = _lane_bcast(ex, h)
                    sbuf[e, pl.ds(h * 16, 16)] = (
                        hsb[e, pl.ds(h * 16, 16)] * bb)
        pltpu.async_copy(sbuf, accn.at[didx], semd, add=True)
        pltpu.async_copy(exbuf, accd.at[dpack], semd, add=True)

    fire_hs(0, 0)
    fire_vm(0)

    def body2(gg, carry):
        g0 = gg * 2
        wait_g(0)
        @pl.when(gg > 0)
        def _():
            drain_s(1)
        fire_hs(1, g0 + 1)
        compute_scatter(0)
        fire_vm(1)
        wait_g(1)
        drain_s(0)
        @pl.when(g0 + 2 < NCHUNK)
        def _():
            fire_hs(0, g0 + 2)
        compute_scatter(1)
        @pl.when(g0 + 2 < NCHUNK)
        def _():
            fire_vm(0)
        return carry
    lax.fori_loop(0, NCHUNK // 2, body2, 0)
    wait_g(0)
    drain_s(1)
    compute_scatter(0)
    drain_s(0)

    plsc.subcore_barrier()
    _strided_units(s, NZN, lambda u: pltpu.sync_copy(
        accn.at[pl.ds(u * ZU, ZU)], num_hbm.at[pl.ds(u * ZU, ZU)]))
    pltpu.sync_copy(accd.at[pl.ds(s * CH, CH)],
                    den_hbm.at[pl.ds(s * CH, CH)])


@functools.cache
def _sc_kernels():
    mesh = plsc.VectorSubcoreMesh(
        core_axis_name="c", subcore_axis_name="s",
        num_cores=NCORE, num_subcores=NSUB)
    deg = pl.kernel(
        _sc_deg_body,
        out_type=jax.ShapeDtypeStruct((NPACKD, DD), _f32),
        mesh=mesh,
        scratch_types=[
            pltpu.VMEM((KD,), _i32),
            pltpu.VMEM((KD,), _i32),
            pltpu.VMEM((KD, DD), _f32),
            pltpu.VMEM_SHARED((NPACKD, DD), _f32),
        ])
    edge = pl.kernel(
        _sc_edge_body,
        out_type=[jax.ShapeDtypeStruct((NN, DD), _f32),
                  jax.ShapeDtypeStruct((NPACK, DD), _f32)],
        mesh=mesh,
        scratch_types=[
            pltpu.VMEM((K,), _i32),
            pltpu.VMEM((K,), _i32),
            pltpu.VMEM((K,), _i32),
            pltpu.VMEM((K,), _i32),
            pltpu.VMEM((K,), _i32),
            pltpu.VMEM((K,), _i32),
            pltpu.VMEM((K, 2 * DD), _f32),
            pltpu.VMEM((K, 2 * DD), _f32),
            pltpu.VMEM((K, DD), _f32),
            pltpu.VMEM((K, DD), _f32),
            pltpu.VMEM((K, DD), _f32),
            pltpu.VMEM_SHARED((NN, DD), _f32),
            pltpu.VMEM_SHARED((NPACK, DD), _f32),
            pltpu.SemaphoreType.DMA,
            pltpu.SemaphoreType.DMA,
            pltpu.SemaphoreType.DMA,
            pltpu.SemaphoreType.DMA,
        ])
    return deg, edge


BN = 1000
GRID = NN // BN


def _a0_body(x_ref, w_ref, dinv_ref, o_ref):
    h1 = jnp.dot(x_ref[...], w_ref[...],
                 preferred_element_type=_f32) * dinv_ref[...]
    o_ref[...] = jnp.concatenate([h1, jnp.zeros_like(h1)], axis=1)


def _c_body(p0, h1p, dinv, bgr, wa, ms, md, hso, eso, edo, mo):
    hg = jnp.maximum(
        (p0[...] + h1p[...][:, :DD]) * dinv[...] + bgr[...], 0.0)
    h2 = jnp.dot(hg, wa[...], preferred_element_type=_f32)
    es = jnp.dot(h2, ms[...], preferred_element_type=_f32)
    ed = jnp.dot(h2, md[...], preferred_element_type=_f32)
    hso[...] = jnp.concatenate(
        [h2, es, es, jnp.zeros((h2.shape[0], 2 * DD - DD - 2 * HH), _f32)],
        axis=1)
    eso[...] = es
    edo[...] = ed
    mo[...] = jnp.max(es, axis=0, keepdims=True)[None]


def _e_common(n0, d0, hs, es, ed, mg, rr, ss, bar):
    esv = es[...]
    edv = ed[...]
    z = esv + edv
    lr = jnp.where(z > 0, z, 0.2 * z)
    mp = jnp.maximum(0.0, mg[...] + edv)
    exs = jnp.exp(lr - mp)
    den = d0[...][:, :HH] + exs
    num = n0[...] + jnp.dot(exs, rr[...],
                            preferred_element_type=_f32) * hs[...][:, :DD]
    denw = jnp.dot(den, rr[...], preferred_element_type=_f32) + 1e-16
    return jnp.dot(num / denw, ss[...], preferred_element_type=_f32) + bar[...]


def _e_body(n0, d0, h2, es, ed, mg, rr, ss, bar, wg, bvec, scale, o_ref):
    o16 = _e_common(n0, d0, h2, es, ed, mg, rr, ss, bar)
    h1 = (jnp.dot(o16, wg[...], preferred_element_type=_f32) * scale[...]
          + bvec[...])
    o_ref[...] = jnp.concatenate([h1, jnp.zeros_like(h1)], axis=1)


def _row_spec(w):
    return pl.BlockSpec((BN, w), lambda i: (i, 0))


def _full_spec(h, w):
    return pl.BlockSpec((h, w), lambda i: (0, 0))


_a0_call = pl.pallas_call(
    _a0_body,
    grid=(GRID,),
    in_specs=[_row_spec(DD), _full_spec(DD, DD), _row_spec(1)],
    out_specs=_row_spec(2 * DD),
    out_shape=jax.ShapeDtypeStruct((NN, 2 * DD), _f32),
)

_c_call = pl.pallas_call(
    _c_body,
    grid=(GRID,),
    in_specs=[_row_spec(DD), _row_spec(2 * DD), _row_spec(1),
              _full_spec(1, DD), _full_spec(DD, DD),
              _full_spec(DD, HH), _full_spec(DD, HH)],
    out_specs=[_row_spec(2 * DD), _row_spec(HH), _row_spec(HH),
               pl.BlockSpec((1, 1, HH), lambda i: (i, 0, 0))],
    out_shape=[jax.ShapeDtypeStruct((NN, 2 * DD), _f32),
               jax.ShapeDtypeStruct((NN, HH), _f32),
               jax.ShapeDtypeStruct((NN, HH), _f32),
               jax.ShapeDtypeStruct((GRID, 1, HH), _f32)],
)

_e_call = pl.pallas_call(
    _e_body,
    grid=(GRID,),
    in_specs=[_row_spec(DD), _row_spec(16),
              _row_spec(2 * DD), _row_spec(HH), _row_spec(HH),
              _full_spec(1, HH), _full_spec(HH, DD), _full_spec(DD, CC),
              _full_spec(1, CC), _full_spec(CC, DD), _full_spec(1, DD),
              _row_spec(1)],
    out_specs=_row_spec(2 * DD),
    out_shape=jax.ShapeDtypeStruct((NN, 2 * DD), _f32),
)


def kernel(x, edge_index, Wg0, Wg12, bg, Wa, a_src, a_dst, ba, Wout, bout):
    src = edge_index[0]
    dst = edge_index[1]

    sc_deg, sc_edge = _sc_kernels()
    degp = sc_deg(dst)
    deg = degp.reshape(NPACKD * 16, 8)[:NN, 0] + 1.0
    dinv = lax.rsqrt(deg)[:, None]

    eye8 = jnp.eye(HH, dtype=_f32)
    rr = jnp.repeat(eye8, CC, axis=1)
    ss = jnp.tile(jnp.eye(CC, dtype=_f32), (HH, 1)) / HH

    z1d = jnp.zeros((1, DD), _f32)
    zdd = jnp.zeros((DD, DD), _f32)
    zdh = jnp.zeros((DD, HH), _f32)
    z1h = jnp.zeros((1, CC), _f32)
    zcd = jnp.zeros((CC, DD), _f32)
    ms_l = [(a_src[i][:, :, None] * eye8[:, None, :]).reshape(DD, HH)
            for i in range(3)]
    md_l = [(a_dst[i][:, :, None] * eye8[:, None, :]).reshape(DD, HH)
            for i in range(3)]
    wout_p = jnp.pad(Wout, ((0, 0), (0, DD - NC7)))
    bout_p = jnp.pad(bout, (0, DD - NC7))[None]
    bg6 = jnp.stack([bg[0][None], z1d, bg[1][None], z1d, bg[2][None], z1d])
    wa6 = jnp.stack([Wa[0], zdd, Wa[1], zdd, Wa[2], zdd])
    ms6 = jnp.stack([ms_l[0], zdh, ms_l[1], zdh, ms_l[2], zdh])
    md6 = jnp.stack([md_l[0], zdh, md_l[1], zdh, md_l[2], zdh])
    ba6 = jnp.stack([z1h, ba[0][None], z1h, ba[1][None], z1h, ba[2][None]])
    wn6 = jnp.stack([zcd, Wg12[0], zcd, Wg12[1], zcd, wout_p])
    bo6 = jnp.stack([z1d, z1d, z1d, z1d, z1d, bout_p])
    parity6 = jnp.arange(6, dtype=_i32) % 2
    last6 = jnp.arange(6, dtype=_i32) == 5

    vmz = jnp.zeros((NN, DD), _f32)
    hs0 = _a0_call(x, Wg0, dinv)
    init = (hs0, vmz, jnp.zeros((NN, HH), _f32), jnp.zeros((NN, HH), _f32),
            jnp.zeros((1, HH), _f32))

    def body(carry, xs):
        hs, vm, es, ed, mg = carry
        bgi, wai, msi, mdi, bai, wni, boi, par, lastf = xs
        nump, denp = sc_edge(hs, vm, src, dst)

        def c_branch(_):
            hs2, es2, ed2, mparts = _c_call(nump, hs, dinv, bgi, wai,
                                            msi, mdi)
            mg2 = jnp.max(mparts[:, 0, :], axis=0, keepdims=True)
            mprime = jnp.maximum(0.0, mg2 + ed2)
            vm2 = jnp.concatenate(
                [ed2, ed2, mprime, mprime,
                 jnp.zeros((NN, DD - 4 * HH), _f32)], axis=1)
            return hs2, vm2, es2, ed2, mg2

        def e_branch(_):
            sc = jnp.where(lastf, jnp.ones_like(dinv), dinv)
            den16 = denp.reshape(NPACK * 8, 16)[:NN]
            hs2 = _e_call(nump, den16, hs, es, ed, mg, rr, ss, bai, wni,
                          boi, sc)
            return hs2, vmz, es, ed, mg

        return lax.cond(par == 0, c_branch, e_branch, 0), 0.0

    (hsf, _, _, _, _), _ = lax.scan(
        body, init, (bg6, wa6, ms6, md6, ba6, wn6, bo6, parity6, last6))
    return hsf[:, :NC7]


# speedup vs baseline: 13.8974x; 1.0393x over previous
import functools

import jax
import jax.numpy as jnp
from jax import lax
from jax.experimental import pallas as pl
from jax.experimental.pallas import tpu as pltpu
from jax.experimental.pallas import tpu_sc as plsc

NN = 10000
NC7 = 7
EE = 320000
DD = 128
HH = 8
CC = 16
NCORE = 1
NSUB = 16
NWORK = NCORE * NSUB
EPW = EE // NWORK
K = 32
NCHUNK = EPW // K
NPACKD = 640
KD = 32
NCHUNKD = EPW // KD
CH = 80
ZU = 16
NZN = NN // ZU
NPACK = 1280

_f32 = jnp.float32
_i32 = jnp.int32

_GDN = lax.GatherDimensionNumbers(
    offset_dims=(), collapsed_slice_dims=(0,), start_index_map=(0,))


def _lane_bcast(v, h):
    idx = jnp.full((16, 1), h, _i32)
    return lax.gather(v, idx, _GDN, (1,),
                      mode=lax.GatherScatterMode.PROMISE_IN_BOUNDS)


def _zero_rows(zb, width, nrows):
    def body(i, carry):
        for j in range(width // 16):
            zb[i, pl.ds(j * 16, 16)] = jnp.zeros((16,), _f32)
        return carry
    lax.fori_loop(0, nrows, body, 0)


def _strided_units(s, nunits, fn):
    nper = -(-nunits // NSUB)

    def body(t, carry):
        cid = s + NSUB * t
        @pl.when(cid < nunits)
        def _():
            fn(cid)
        return carry
    lax.fori_loop(0, nper, body, 0)


def _wid(c, s):
    return c * NSUB + s


def _sc_deg_body(dst_hbm, out_hbm, didx, dpack, ones, acc):
    c = lax.axis_index("c")
    s = lax.axis_index("s")
    _zero_rows(ones, DD, KD)

    lane = lax.iota(_i32, 16)
    one_a = jnp.where(lane == 0, 1.0, 0.0).astype(_f32)
    one_b = jnp.where(lane == 8, 1.0, 0.0).astype(_f32)
    zrow = jnp.zeros((16,), _f32)

    _strided_units(s, NPACKD // ZU, lambda u: pltpu.sync_copy(
        ones.at[pl.ds(0, ZU)], acc.at[pl.ds(u * ZU, ZU)]))
    plsc.subcore_barrier()

    base0 = _wid(c, s) * EPW

    def chunk(g, carry):
        pltpu.sync_copy(dst_hbm.at[pl.ds(base0 + g * KD, KD)], didx)

        def grp(v, carry2):
            dv = didx[pl.ds(v * 16, 16)]
            dpack[pl.ds(v * 16, 16)] = lax.shift_right_logical(dv, 4)
            q = dv & 15
            offv = (q & 14) * 8
            pf = (q & 1).astype(_f32)
            for l in range(16):
                e = v * 16 + l
                for j in range(DD // 16):
                    ones[e, pl.ds(j * 16, 16)] = zrow
                ones[e, pl.ds(offv[l], 16)] = (one_b * pf[l]
                                               + one_a * (1.0 - pf[l]))
            return carry2
        lax.fori_loop(0, KD // 16, grp, 0)

        pltpu.sync_copy(ones, acc.at[dpack], add=True)
        return carry
    lax.fori_loop(0, NCHUNKD, chunk, 0)

    plsc.subcore_barrier()
    pltpu.sync_copy(acc.at[pl.ds(s * (NPACKD // NSUB), NPACKD // NSUB)],
                    out_hbm.at[pl.ds(s * (NPACKD // NSUB), NPACKD // NSUB)])


def _sc_edge_body(hs_hbm, vm_hbm, src_hbm, dst_hbm, num_hbm, den_hbm,
                  sidx0, didx0, dpack0, sidx1, didx1, dpack1,
                  hsbuf0, hsbuf1, vmbuf, sbuf, exbuf,
                  accn, accd, semg0, semg1, semv, semd):
    c = lax.axis_index("c")
    s = lax.axis_index("s")
    _zero_rows(sbuf, DD, K)

    _strided_units(s, NZN, lambda u: pltpu.sync_copy(
        sbuf.at[pl.ds(0, ZU)], accn.at[pl.ds(u * ZU, ZU)]))
    _strided_units(s, NPACK // ZU, lambda u: pltpu.sync_copy(
        sbuf.at[pl.ds(0, ZU)], accd.at[pl.ds(u * ZU, ZU)]))
    plsc.subcore_barrier()

    base0 = _wid(c, s) * EPW
    zrow = jnp.zeros((16,), _f32)
    sets = ((sidx0, didx0, dpack0, hsbuf0, semg0),
            (sidx1, didx1, dpack1, hsbuf1, semg1))

    def fire_hs(b, g):
        sidx, didx, _, hsb, sg = sets[b]
        base = base0 + g * K
        pltpu.sync_copy(src_hbm.at[pl.ds(base, K)], sidx)
        pltpu.sync_copy(dst_hbm.at[pl.ds(base, K)], didx)
        pltpu.async_copy(hs_hbm.at[sidx], hsb, sg)

    def fire_vm(b):
        pltpu.async_copy(vm_hbm.at[sets[b][1]], vmbuf, semv)

    def wait_g(b):
        sidx, didx, _, hsb, sg = sets[b]
        pltpu.make_async_copy(hs_hbm.at[sidx], hsb, sg).wait()
        pltpu.make_async_copy(vm_hbm.at[didx], vmbuf, semv).wait()

    def drain_s(b):
        _, didx, dpack, _, _ = sets[b]
        pltpu.make_async_copy(sbuf, accn.at[didx], semd).wait()
        pltpu.make_async_copy(exbuf, accd.at[dpack], semd).wait()

    def compute_scatter(b):
        _, didx, dpack, hsb, _ = sets[b]
        for v in range(K // 16):
            dv = didx[pl.ds(v * 16, 16)]
            dpack[pl.ds(v * 16, 16)] = lax.shift_right_logical(dv, 3)
            offv = (dv & 7) * 16
            for l in range(16):
                e = v * 16 + l
                off = offv[l]
                u = hsb[e, pl.ds(DD, 16)]
                edv = vmbuf[e, pl.ds(0, 16)]
                mm = vmbuf[e, pl.ds(16, 16)]
                z = u + edv
                lr = jnp.where(z > 0, z, 0.2 * z)
                ex = jnp.exp(lr - mm)
                for j in range(DD // 16):
                    exbuf[e, pl.ds(j * 16, 16)] = zrow
                exbuf[e, pl.ds(off, 16)] = ex
                for h in range(HH):
                    bb = _lane_bcast(ex, h)
                    sbuf[e, pl.ds(h * 16, 16)] = (
                        hsb[e, pl.ds(h * 16, 16)] * bb)
        pltpu.async_copy(sbuf, accn.at[didx], semd, add=True)
        pltpu.async_copy(exbuf, accd.at[dpack], semd, add=True)

    fire_hs(0, 0)
    fire_vm(0)

    def body2(gg, carry):
        g0 = gg * 2
        wait_g(0)
        @pl.when(gg > 0)
        def _():
            drain_s(1)
        fire_hs(1, g0 + 1)
        compute_scatter(0)
        fire_vm(1)
        wait_g(1)
        drain_s(0)
        @pl.when(g0 + 2 < NCHUNK)
        def _():
            fire_hs(0, g0 + 2)
        compute_scatter(1)
        @pl.when(g0 + 2 < NCHUNK)
        def _():
            fire_vm(0)
        return carry
    lax.fori_loop(0, NCHUNK // 2, body2, 0)
    wait_g(0)
    drain_s(1)
    compute_scatter(0)
    drain_s(0)

    plsc.subcore_barrier()

    def copy80(t, carry):
        cid = s + NSUB * t
        @pl.when(cid < NN // CH)
        def _():
            pltpu.sync_copy(accn.at[pl.ds(cid * CH, CH)],
                            num_hbm.at[pl.ds(cid * CH, CH)])
        return carry
    lax.fori_loop(0, -(-(NN // CH) // NSUB), copy80, 0)
    pltpu.sync_copy(accd.at[pl.ds(s * CH, CH)],
                    den_hbm.at[pl.ds(s * CH, CH)])


@functools.cache
def _sc_kernels():
    mesh = plsc.VectorSubcoreMesh(
        core_axis_name="c", subcore_axis_name="s",
        num_cores=NCORE, num_subcores=NSUB)
    deg = pl.kernel(
        _sc_deg_body,
        out_type=jax.ShapeDtypeStruct((NPACKD, DD), _f32),
        mesh=mesh,
        scratch_types=[
            pltpu.VMEM((KD,), _i32),
            pltpu.VMEM((KD,), _i32),
            pltpu.VMEM((KD, DD), _f32),
            pltpu.VMEM_SHARED((NPACKD, DD), _f32),
        ])
    edge = pl.kernel(
        _sc_edge_body,
        out_type=[jax.ShapeDtypeStruct((NN, DD), _f32),
                  jax.ShapeDtypeStruct((NPACK, DD), _f32)],
        mesh=mesh,
        scratch_types=[
            pltpu.VMEM((K,), _i32),
            pltpu.VMEM((K,), _i32),
            pltpu.VMEM((K,), _i32),
            pltpu.VMEM((K,), _i32),
            pltpu.VMEM((K,), _i32),
            pltpu.VMEM((K,), _i32),
            pltpu.VMEM((K, 2 * DD), _f32),
            pltpu.VMEM((K, 2 * DD), _f32),
            pltpu.VMEM((K, DD), _f32),
            pltpu.VMEM((K, DD), _f32),
            pltpu.VMEM((K, DD), _f32),
            pltpu.VMEM_SHARED((NN, DD), _f32),
            pltpu.VMEM_SHARED((NPACK, DD), _f32),
            pltpu.SemaphoreType.DMA,
            pltpu.SemaphoreType.DMA,
            pltpu.SemaphoreType.DMA,
            pltpu.SemaphoreType.DMA,
        ])
    return deg, edge


BN = 1000
GRID = NN // BN


def _a0_body(x_ref, w_ref, dinv_ref, o_ref):
    h1 = jnp.dot(x_ref[...], w_ref[...],
                 preferred_element_type=_f32) * dinv_ref[...]
    o_ref[...] = jnp.concatenate([h1, jnp.zeros_like(h1)], axis=1)


def _c_body(p0, h1p, dinv, bgr, wa, ms, md, hso, eso, edo, mo):
    hg = jnp.maximum(
        (p0[...] + h1p[...][:, :DD]) * dinv[...] + bgr[...], 0.0)
    h2 = jnp.dot(hg, wa[...], preferred_element_type=_f32)
    es = jnp.dot(h2, ms[...], preferred_element_type=_f32)
    ed = jnp.dot(h2, md[...], preferred_element_type=_f32)
    hso[...] = jnp.concatenate(
        [h2, es, es, jnp.zeros((h2.shape[0], 2 * DD - DD - 2 * HH), _f32)],
        axis=1)
    eso[...] = es
    edo[...] = ed
    mo[...] = jnp.max(es, axis=0, keepdims=True)[None]


def _e_common(n0, d0, hs, es, ed, mg, rr, ss, bar):
    esv = es[...]
    edv = ed[...]
    z = esv + edv
    lr = jnp.where(z > 0, z, 0.2 * z)
    mp = jnp.maximum(0.0, mg[...] + edv)
    exs = jnp.exp(lr - mp)
    den = d0[...][:, :HH] + exs
    num = n0[...] + jnp.dot(exs, rr[...],
                            preferred_element_type=_f32) * hs[...][:, :DD]
    denw = jnp.dot(den, rr[...], preferred_element_type=_f32) + 1e-16
    return jnp.dot(num / denw, ss[...], preferred_element_type=_f32) + bar[...]


def _e_body(n0, d0, h2, es, ed, mg, rr, ss, bar, wg, bvec, scale, o_ref):
    o16 = _e_common(n0, d0, h2, es, ed, mg, rr, ss, bar)
    h1 = (jnp.dot(o16, wg[...], preferred_element_type=_f32) * scale[...]
          + bvec[...])
    o_ref[...] = jnp.concatenate([h1, jnp.zeros_like(h1)], axis=1)


def _row_spec(w):
    return pl.BlockSpec((BN, w), lambda i: (i, 0))


def _full_spec(h, w):
    return pl.BlockSpec((h, w), lambda i: (0, 0))


_a0_call = pl.pallas_call(
    _a0_body,
    grid=(GRID,),
    in_specs=[_row_spec(DD), _full_spec(DD, DD), _row_spec(1)],
    out_specs=_row_spec(2 * DD),
    out_shape=jax.ShapeDtypeStruct((NN, 2 * DD), _f32),
)

_c_call = pl.pallas_call(
    _c_body,
    grid=(GRID,),
    in_specs=[_row_spec(DD), _row_spec(2 * DD), _row_spec(1),
              _full_spec(1, DD), _full_spec(DD, DD),
              _full_spec(DD, HH), _full_spec(DD, HH)],
    out_specs=[_row_spec(2 * DD), _row_spec(HH), _row_spec(HH),
               pl.BlockSpec((1, 1, HH), lambda i: (i, 0, 0))],
    out_shape=[jax.ShapeDtypeStruct((NN, 2 * DD), _f32),
               jax.ShapeDtypeStruct((NN, HH), _f32),
               jax.ShapeDtypeStruct((NN, HH), _f32),
               jax.ShapeDtypeStruct((GRID, 1, HH), _f32)],
)

_e_call = pl.pallas_call(
    _e_body,
    grid=(GRID,),
    in_specs=[_row_spec(DD), _row_spec(16),
              _row_spec(2 * DD), _row_spec(HH), _row_spec(HH),
              _full_spec(1, HH), _full_spec(HH, DD), _full_spec(DD, CC),
              _full_spec(1, CC), _full_spec(CC, DD), _full_spec(1, DD),
              _row_spec(1)],
    out_specs=_row_spec(2 * DD),
    out_shape=jax.ShapeDtypeStruct((NN, 2 * DD), _f32),
)


def kernel(x, edge_index, Wg0, Wg12, bg, Wa, a_src, a_dst, ba, Wout, bout):
    src = edge_index[0]
    dst = edge_index[1]

    sc_deg, sc_edge = _sc_kernels()
    degp = sc_deg(dst)
    deg = degp.reshape(NPACKD * 16, 8)[:NN, 0] + 1.0
    dinv = lax.rsqrt(deg)[:, None]

    eye8 = jnp.eye(HH, dtype=_f32)
    rr = jnp.repeat(eye8, CC, axis=1)
    ss = jnp.tile(jnp.eye(CC, dtype=_f32), (HH, 1)) / HH

    z1d = jnp.zeros((1, DD), _f32)
    zdd = jnp.zeros((DD, DD), _f32)
    zdh = jnp.zeros((DD, HH), _f32)
    z1h = jnp.zeros((1, CC), _f32)
    zcd = jnp.zeros((CC, DD), _f32)
    ms_l = [(a_src[i][:, :, None] * eye8[:, None, :]).reshape(DD, HH)
            for i in range(3)]
    md_l = [(a_dst[i][:, :, None] * eye8[:, None, :]).reshape(DD, HH)
            for i in range(3)]
    wout_p = jnp.pad(Wout, ((0, 0), (0, DD - NC7)))
    bout_p = jnp.pad(bout, (0, DD - NC7))[None]
    bg6 = jnp.stack([bg[0][None], z1d, bg[1][None], z1d, bg[2][None], z1d])
    wa6 = jnp.stack([Wa[0], zdd, Wa[1], zdd, Wa[2], zdd])
    ms6 = jnp.stack([ms_l[0], zdh, ms_l[1], zdh, ms_l[2], zdh])
    md6 = jnp.stack([md_l[0], zdh, md_l[1], zdh, md_l[2], zdh])
    ba6 = jnp.stack([z1h, ba[0][None], z1h, ba[1][None], z1h, ba[2][None]])
    wn6 = jnp.stack([zcd, Wg12[0], zcd, Wg12[1], zcd, wout_p])
    bo6 = jnp.stack([z1d, z1d, z1d, z1d, z1d, bout_p])
    parity6 = jnp.arange(6, dtype=_i32) % 2
    last6 = jnp.arange(6, dtype=_i32) == 5

    vmz = jnp.zeros((NN, DD), _f32)
    hs0 = _a0_call(x, Wg0, dinv)
    init = (hs0, vmz, jnp.zeros((NN, HH), _f32), jnp.zeros((NN, HH), _f32),
            jnp.zeros((1, HH), _f32))

    def body(carry, xs):
        hs, vm, es, ed, mg = carry
        bgi, wai, msi, mdi, bai, wni, boi, par, lastf = xs
        nump, denp = sc_edge(hs, vm, src, dst)

        def c_branch(_):
            hs2, es2, ed2, mparts = _c_call(nump, hs, dinv, bgi, wai,
                                            msi, mdi)
            mg2 = jnp.max(mparts[:, 0, :], axis=0, keepdims=True)
            mprime = jnp.maximum(0.0, mg2 + ed2)
            vm2 = jnp.concatenate(
                [ed2, ed2, mprime, mprime,
                 jnp.zeros((NN, DD - 4 * HH), _f32)], axis=1)
            return hs2, vm2, es2, ed2, mg2

        def e_branch(_):
            sc = jnp.where(lastf, jnp.ones_like(dinv), dinv)
            den16 = denp.reshape(NPACK * 8, 16)[:NN]
            hs2 = _e_call(nump, den16, hs, es, ed, mg, rr, ss, bai, wni,
                          boi, sc)
            return hs2, vmz, es, ed, mg

        return lax.cond(par == 0, c_branch, e_branch, 0), 0.0

    (hsf, _, _, _, _), _ = lax.scan(
        body, init, (bg6, wa6, ms6, md6, ba6, wn6, bo6, parity6, last6))
    return hsf[:, :NC7]
